# baseline (speedup 1.0000x reference)
import jax
import jax.numpy as jnp
from jax import lax
from jax.experimental import pallas as pl
from jax.experimental.pallas import tpu as pltpu


def _cdiv(a, b):
    return -(-a // b)


def _subsample_kernel(x_ref, o_ref):
    _, ct, ho, wo = o_ref.shape
    w = x_ref.shape[-1]
    xv = x_ref[0, :, pl.ds(0, ho, stride=2), :]
    rows = lax.broadcasted_iota(jnp.int32, (w, wo), 0)
    cols = lax.broadcasted_iota(jnp.int32, (w, wo), 1)
    sel = (rows == 2 * cols).astype(x_ref.dtype)
    out = jnp.dot(xv.reshape(ct * ho, w), sel,
                  preferred_element_type=jnp.float32)
    o_ref[...] = out.reshape(1, ct, ho, wo).astype(o_ref.dtype)


def kernel(x):
    n, c, h, w = x.shape
    ho = (h - 1) // 2 + 1
    wo = (w - 1) // 2 + 1

    ct = 64
    grid = (n, _cdiv(c, ct))

    out = pl.pallas_call(
        _subsample_kernel,
        out_shape=jax.ShapeDtypeStruct((n, c, ho, wo), x.dtype),
        grid=grid,
        in_specs=[pl.BlockSpec((1, ct, h, w), lambda i, j: (i, j, 0, 0))],
        out_specs=pl.BlockSpec((1, ct, ho, wo), lambda i, j: (i, j, 0, 0)),
        compiler_params=pltpu.CompilerParams(
            dimension_semantics=("parallel", "parallel")),
        cost_estimate=pl.CostEstimate(
            flops=2 * n * c * ho * w * wo, transcendentals=0,
            bytes_accessed=(n * c * h * w + n * c * ho * wo) * x.dtype.itemsize),
    )(x)
    return [out]


# speedup vs baseline: 1.7465x; 1.7465x over previous
import jax
import jax.numpy as jnp
from jax import lax
from jax.experimental import pallas as pl
from jax.experimental.pallas import tpu as pltpu


def _cdiv(a, b):
    return -(-a // b)


def _subsample_kernel(x_ref, o_ref):
    _, ct, hp, wp = x_ref.shape
    rp = o_ref.shape[2]
    wo = o_ref.shape[3] // 4
    row = lax.broadcasted_iota(jnp.int32, (wp, 4 * wo), 0)
    col = lax.broadcasted_iota(jnp.int32, (wp, 4 * wo), 1)
    acc = jnp.zeros((ct * rp, 4 * wo), jnp.float32)
    for q in range(4):
        xq = x_ref[0, :, pl.ds(q, rp, stride=4), :].reshape(ct * rp, wp)
        grp = jnp.logical_and(col >= wo * q, col < wo * (q + 1))
        sel = jnp.where(jnp.logical_and(grp, row == 2 * (col - wo * q)),
                        1.0, 0.0).astype(x_ref.dtype)
        acc = acc + jnp.dot(xq, sel, preferred_element_type=jnp.float32)
    o_ref[...] = acc.reshape(1, ct, rp, 4 * wo).astype(o_ref.dtype)


def kernel(x):
    n, c, h, w = x.shape
    ho, wo = h // 2, w // 2

    xp = x.reshape(n, c, h // 2, 2 * w)

    ct = 64
    grid = (n, _cdiv(c, ct))

    out = pl.pallas_call(
        _subsample_kernel,
        out_shape=jax.ShapeDtypeStruct((n, c, ho // 4, 4 * wo), x.dtype),
        grid=grid,
        in_specs=[pl.BlockSpec((1, ct, h // 2, 2 * w),
                               lambda i, j: (i, j, 0, 0))],
        out_specs=pl.BlockSpec((1, ct, ho // 4, 4 * wo),
                               lambda i, j: (i, j, 0, 0)),
        compiler_params=pltpu.CompilerParams(
            dimension_semantics=("parallel", "parallel")),
        cost_estimate=pl.CostEstimate(
            flops=2 * n * c * ho * w * wo * 4, transcendentals=0,
            bytes_accessed=(n * c * h * w + n * c * ho * wo) * x.dtype.itemsize),
    )(xp)
    return [out.reshape(n, c, ho, wo)]


# speedup vs baseline: 5.5278x; 3.1650x over previous
import jax
import jax.numpy as jnp
from jax.experimental import pallas as pl
from jax.experimental.pallas import tpu as pltpu


def _cdiv(a, b):
    return -(-a // b)


def _subsample_kernel(x_ref, o_ref):
    wo = o_ref.shape[1]
    o_ref[...] = x_ref[:, 0, pl.ds(0, wo, stride=2), :]


def kernel(x):
    n, c, h, w = x.shape
    ho, wo = h // 2, w // 2

    xt = jnp.transpose(x, (0, 2, 3, 1)).reshape(n * ho, 2, w, c)

    bp = 16
    ctile = 128
    grid = (_cdiv(n * ho, bp), c // ctile)

    out = pl.pallas_call(
        _subsample_kernel,
        out_shape=jax.ShapeDtypeStruct((n * ho, wo, c), x.dtype),
        grid=grid,
        in_specs=[pl.BlockSpec((bp, 1, w, ctile),
                               lambda i, j: (i, 0, 0, j))],
        out_specs=pl.BlockSpec((bp, wo, ctile), lambda i, j: (i, 0, j)),
        compiler_params=pltpu.CompilerParams(
            dimension_semantics=("parallel", "parallel")),
        cost_estimate=pl.CostEstimate(
            flops=0, transcendentals=0,
            bytes_accessed=(n * ho * w * c + n * ho * wo * c) * x.dtype.itemsize),
    )(xt)
    return [jnp.transpose(out.reshape(n, ho, wo, c), (0, 3, 1, 2))]


# speedup vs baseline: 8.3317x; 1.5072x over previous
import jax
import jax.numpy as jnp
from jax.experimental import pallas as pl
from jax.experimental.pallas import tpu as pltpu


def _cdiv(a, b):
    return -(-a // b)


def _subsample_kernel(x_ref, o_ref):
    wo = o_ref.shape[1]
    o_ref[...] = x_ref[:, 0, pl.ds(0, wo, stride=2), :]


def kernel(x):
    n, c, h, w = x.shape
    ho, wo = h // 2, w // 2

    xt = jnp.transpose(x, (0, 2, 3, 1)).reshape(n * ho, 2, w, c)

    bp = 32
    ctile = 128
    grid = (_cdiv(n * ho, bp), c // ctile)

    out = pl.pallas_call(
        _subsample_kernel,
        out_shape=jax.ShapeDtypeStruct((n * ho, wo, c), x.dtype),
        grid=grid,
        in_specs=[pl.BlockSpec((bp, 1, w, ctile),
                               lambda i, j: (i, 0, 0, j))],
        out_specs=pl.BlockSpec((bp, wo, ctile), lambda i, j: (i, 0, j)),
        compiler_params=pltpu.CompilerParams(
            dimension_semantics=("parallel", "parallel")),
        cost_estimate=pl.CostEstimate(
            flops=0, transcendentals=0,
            bytes_accessed=(n * ho * w * c + n * ho * wo * c) * x.dtype.itemsize),
    )(xt)
    return [jnp.transpose(out.reshape(n, ho, wo, c), (0, 3, 1, 2))]


# speedup vs baseline: 11.2134x; 1.3459x over previous
import jax
import jax.numpy as jnp
from jax.experimental import pallas as pl
from jax.experimental.pallas import tpu as pltpu


def _cdiv(a, b):
    return -(-a // b)


def _subsample_kernel(x_ref, o_ref):
    wo = o_ref.shape[1]
    o_ref[...] = x_ref[:, 0, pl.ds(0, wo, stride=2), :]


def kernel(x):
    n, c, h, w = x.shape
    ho, wo = h // 2, w // 2

    xt = jnp.transpose(x, (0, 2, 3, 1)).reshape(n * ho, 2, w, c)

    bp = 64
    ctile = 128
    grid = (_cdiv(n * ho, bp), c // ctile)

    out = pl.pallas_call(
        _subsample_kernel,
        out_shape=jax.ShapeDtypeStruct((n * ho, wo, c), x.dtype),
        grid=grid,
        in_specs=[pl.BlockSpec((bp, 1, w, ctile),
                               lambda i, j: (i, 0, 0, j))],
        out_specs=pl.BlockSpec((bp, wo, ctile), lambda i, j: (i, 0, j)),
        compiler_params=pltpu.CompilerParams(
            dimension_semantics=("parallel", "parallel")),
        cost_estimate=pl.CostEstimate(
            flops=0, transcendentals=0,
            bytes_accessed=(n * ho * w * c + n * ho * wo * c) * x.dtype.itemsize),
    )(xt)
    return [jnp.transpose(out.reshape(n, ho, wo, c), (0, 3, 1, 2))]


# speedup vs baseline: 13.0733x; 1.1659x over previous
import jax
import jax.numpy as jnp
from jax.experimental import pallas as pl
from jax.experimental.pallas import tpu as pltpu


def _cdiv(a, b):
    return -(-a // b)


def _subsample_kernel(x_ref, o_ref):
    wo = o_ref.shape[1]
    o_ref[...] = x_ref[:, 0, pl.ds(0, wo, stride=2), :]


def kernel(x):
    n, c, h, w = x.shape
    ho, wo = h // 2, w // 2

    xt = jnp.transpose(x, (0, 2, 3, 1)).reshape(n * ho, 2, w, c)

    bp = 128
    ctile = 128
    grid = (_cdiv(n * ho, bp), c // ctile)

    out = pl.pallas_call(
        _subsample_kernel,
        out_shape=jax.ShapeDtypeStruct((n * ho, wo, c), x.dtype),
        grid=grid,
        in_specs=[pl.BlockSpec((bp, 1, w, ctile),
                               lambda i, j: (i, 0, 0, j))],
        out_specs=pl.BlockSpec((bp, wo, ctile), lambda i, j: (i, 0, j)),
        compiler_params=pltpu.CompilerParams(
            dimension_semantics=("parallel", "parallel")),
        cost_estimate=pl.CostEstimate(
            flops=0, transcendentals=0,
            bytes_accessed=(n * ho * w * c + n * ho * wo * c) * x.dtype.itemsize),
    )(xt)
    return [jnp.transpose(out.reshape(n, ho, wo, c), (0, 3, 1, 2))]


# speedup vs baseline: 13.3260x; 1.0193x over previous
import jax
import jax.numpy as jnp
from jax.experimental import pallas as pl
from jax.experimental.pallas import tpu as pltpu


def _cdiv(a, b):
    return -(-a // b)


def _subsample_kernel(x_ref, o_ref):
    wo = o_ref.shape[1]
    o_ref[...] = x_ref[:, 0, pl.ds(0, wo, stride=2), :]


def kernel(x):
    n, c, h, w = x.shape
    ho, wo = h // 2, w // 2

    xt = jnp.transpose(x, (0, 2, 3, 1)).reshape(n * ho, 2, w, c)

    bp = 256
    ctile = 128
    grid = (_cdiv(n * ho, bp), c // ctile)

    out = pl.pallas_call(
        _subsample_kernel,
        out_shape=jax.ShapeDtypeStruct((n * ho, wo, c), x.dtype),
        grid=grid,
        in_specs=[pl.BlockSpec((bp, 1, w, ctile),
                               lambda i, j: (i, 0, 0, j))],
        out_specs=pl.BlockSpec((bp, wo, ctile), lambda i, j: (i, 0, j)),
        compiler_params=pltpu.CompilerParams(
            dimension_semantics=("parallel", "parallel")),
        cost_estimate=pl.CostEstimate(
            flops=0, transcendentals=0,
            bytes_accessed=(n * ho * w * c + n * ho * wo * c) * x.dtype.itemsize),
    )(xt)
    return [jnp.transpose(out.reshape(n, ho, wo, c), (0, 3, 1, 2))]
